# baseline (speedup 1.0000x reference)
import functools
import jax
import jax.numpy as jnp
from jax import lax
from jax.experimental import pallas as pl
from jax.experimental.pallas import tpu as pltpu
from jax.experimental.pallas import tpu_sc as plsc

N_NODES = 10000
N_EDGES = 160000
D = 256
DH = 128

ROW_BLOCK = 1000
N_BLOCKS = N_NODES // ROW_BLOCK

EC = 128
E_ROWS = N_EDGES // EC
N_SUB = 16
NP = 10112
ROWS_PER_TILE = NP // N_SUB
CNT_W = 8
NLANE = 16
E_ROWS_P = 1280
CH_PER_TILE = E_ROWS_P // N_SUB
N_PHASE = 2
CH_BLK = CH_PER_TILE // N_PHASE
T_ITER = CH_BLK // 2

_sc_mesh = plsc.VectorSubcoreMesh(core_axis_name="c", subcore_axis_name="s",
                                  num_cores=2, num_subcores=N_SUB)


def _count_body(e2, ones_hbm, zeros_hbm, out_hbm, acc_sh, idx_v, ones_v):
    c = lax.axis_index("c")
    sid = lax.axis_index("s")
    stripe = pl.ds(sid * ROWS_PER_TILE, ROWS_PER_TILE)
    pltpu.sync_copy(ones_hbm, ones_v)
    pltpu.sync_copy(zeros_hbm.at[stripe], acc_sh.at[stripe])
    plsc.subcore_barrier()

    n_iter = (E_ROWS - sid + N_SUB - 1) // N_SUB

    def body(j, carry):
        r = sid + j * N_SUB
        pltpu.sync_copy(e2.at[c, r], idx_v)
        pltpu.sync_copy(ones_v, acc_sh.at[idx_v], add=True)
        return carry

    lax.fori_loop(0, n_iter, body, 0)
    plsc.subcore_barrier()
    pltpu.sync_copy(acc_sh.at[stripe], out_hbm.at[c, stripe])


_count_call = functools.partial(
    pl.kernel,
    out_type=jax.ShapeDtypeStruct((2, NP, CNT_W), jnp.float32),
    mesh=_sc_mesh,
    compiler_params=pltpu.CompilerParams(use_tc_tiling_on_sc=False),
    scratch_types=[
        pltpu.VMEM_SHARED((NP, CNT_W), jnp.float32),
        pltpu.VMEM((EC,), jnp.int32),
        pltpu.VMEM((EC, CNT_W), jnp.float32),
    ],
)(_count_body)


def _spmm_body(gcat, e2p, zeros_hbm, out_cat,
               acc_sh, src_a, dst_a, buf0, buf1,
               sg0, sg1, ss0, ss1):
    c = lax.axis_index("c")
    sid = lax.axis_index("s")
    stripe = pl.ds(sid * ROWS_PER_TILE, ROWS_PER_TILE)
    rbase = sid * CH_PER_TILE
    coff = c * NP
    pltpu.sync_copy(zeros_hbm.at[stripe], acc_sh.at[stripe])
    plsc.subcore_barrier()

    def phase(ph, carry):
        blk = rbase + ph * CH_BLK
        pltpu.sync_copy(e2p.at[0, pl.ds(blk, CH_BLK)], src_a)
        pltpu.sync_copy(e2p.at[1, pl.ds(blk, CH_BLK)], dst_a)

        def idx_body(j, carry2):
            for k in range(EC // NLANE):
                sl = pl.ds(k * NLANE, NLANE)
                src_a[j, sl] = src_a[j, sl] + coff
            return carry2

        lax.fori_loop(0, CH_BLK, idx_body, 0)
        pltpu.async_copy(gcat.at[src_a.at[0]], buf0, sg0)

        def body(t, carry2):
            a = 2 * t
            b = a + 1

            @pl.when(t > 0)
            def _():
                pltpu.make_async_copy(buf1, acc_sh.at[dst_a.at[a - 1]],
                                      ss1).wait()

            pltpu.async_copy(gcat.at[src_a.at[b]], buf1, sg1)
            pltpu.make_async_copy(gcat.at[src_a.at[a]], buf0, sg0).wait()
            pltpu.async_copy(buf0, acc_sh.at[dst_a.at[a]], ss0, add=True)
            pltpu.make_async_copy(gcat.at[src_a.at[b]], buf1, sg1).wait()
            pltpu.make_async_copy(buf0, acc_sh.at[dst_a.at[a]], ss0).wait()

            @pl.when(t < T_ITER - 1)
            def _():
                pltpu.async_copy(gcat.at[src_a.at[a + 2]], buf0, sg0)

            pltpu.async_copy(buf1, acc_sh.at[dst_a.at[b]], ss1, add=True)
            return carry2

        lax.fori_loop(0, T_ITER, body, 0)
        pltpu.make_async_copy(buf1, acc_sh.at[dst_a.at[CH_BLK - 1]],
                              ss1).wait()
        return carry

    lax.fori_loop(0, N_PHASE, phase, 0)
    plsc.subcore_barrier()
    pltpu.sync_copy(acc_sh.at[stripe], out_cat.at[c, stripe])


_spmm_call = functools.partial(
    pl.kernel,
    out_type=jax.ShapeDtypeStruct((2, NP, DH), jnp.float32),
    mesh=_sc_mesh,
    scratch_types=[
        pltpu.VMEM_SHARED((NP, DH), jnp.float32),
        pltpu.VMEM((CH_BLK, EC), jnp.int32),
        pltpu.VMEM((CH_BLK, EC), jnp.int32),
        pltpu.VMEM((EC, DH), jnp.float32),
        pltpu.VMEM((EC, DH), jnp.float32),
        pltpu.SemaphoreType.DMA,
        pltpu.SemaphoreType.DMA,
        pltpu.SemaphoreType.DMA,
        pltpu.SemaphoreType.DMA,
    ],
)(_spmm_body)


def _prep_body(cnt_ref, nf_ref, p_ref, q_ref, g_ref):
    cs = cnt_ref[0, :, 0:1]
    cd = cnt_ref[1, :, 0:1]
    p = jnp.where(cs > 0, lax.rsqrt(cs), 0.0)
    q = jnp.where(cd > 0, lax.rsqrt(cd), 0.0)
    p_ref[...] = jnp.broadcast_to(p, p_ref.shape)
    q_ref[...] = jnp.broadcast_to(q, q_ref.shape)
    g = nf_ref[...] * p
    g_ref[0] = g[:, :DH]
    g_ref[1] = g[:, DH:]


def _mid_body(s_ref, p_ref, q_ref, h_ref, g_ref):
    p = p_ref[:, 0:1]
    q = q_ref[:, 0:1]
    hlo = s_ref[0] * q
    hhi = s_ref[1] * q
    h_ref[...] = jnp.concatenate([hlo, hhi], axis=1)
    g_ref[0] = hlo * p
    g_ref[1] = hhi * p


def _pre_stats_body(h0_ref, h1_ref, s2_ref, q_ref,
                    w0_ref, w1_ref, w2_ref, b_ref,
                    pre_ref, stats_ref, acc_ref):
    i = pl.program_id(0)

    @pl.when(i == 0)
    def _():
        acc_ref[...] = jnp.zeros_like(acc_ref)

    q = q_ref[:, 0:1]
    h2 = jnp.concatenate([s2_ref[0] * q, s2_ref[1] * q], axis=1)
    dn = (((1,), (1,)), ((), ()))
    pre = lax.dot_general(h0_ref[...], w0_ref[...], dn,
                          preferred_element_type=jnp.float32)
    pre += lax.dot_general(h1_ref[...], w1_ref[...], dn,
                           preferred_element_type=jnp.float32)
    pre += lax.dot_general(h2, w2_ref[...], dn,
                           preferred_element_type=jnp.float32)
    pre += b_ref[...]
    pre_ref[...] = pre
    acc_ref[0, :] += jnp.sum(pre, axis=0)
    acc_ref[1, :] += jnp.sum(pre * pre, axis=0)

    @pl.when(i == N_BLOCKS - 1)
    def _():
        stats_ref[...] = acc_ref[...]


def _bn_apply_body(pre_ref, stats_ref, res_ref, g_ref, be_ref, out_ref):
    s = stats_ref[0, :]
    ss = stats_ref[1, :]
    mean = s / N_NODES
    var = ss / N_NODES - mean * mean
    rstd = lax.rsqrt(var + 1e-5)
    h = (pre_ref[...] - mean) * (rstd * g_ref[...]) + be_ref[...]
    out_ref[...] = res_ref[...] + jnp.maximum(h, 0.0)


def _row_spec(w=D):
    return pl.BlockSpec((ROW_BLOCK, w), lambda i: (i, 0))


_cat_spec = pl.BlockSpec((2, ROW_BLOCK, DH), lambda i: (0, i, 0))
_pq_spec = pl.BlockSpec((ROW_BLOCK, CNT_W), lambda i: (i, 0))
_cat_shape = jax.ShapeDtypeStruct((2, NP, DH), jnp.float32)
_pq_shape = jax.ShapeDtypeStruct((N_NODES, CNT_W), jnp.float32)


@jax.jit
def kernel(node_feat, edge_index, W0, W1, W2, bias, gamma, beta):
    e2 = edge_index.reshape(2, E_ROWS, EC)
    n_pad = E_ROWS_P * EC - N_EDGES
    src_p = jnp.concatenate(
        [edge_index[0], jnp.zeros((n_pad,), jnp.int32)])
    dst_p = jnp.concatenate(
        [edge_index[1], jnp.full((n_pad,), NP - 1, jnp.int32)])
    e2p = jnp.stack([src_p, dst_p]).reshape(2, E_ROWS_P, EC)
    ones_c = jnp.ones((EC, CNT_W), jnp.float32)
    zeros_c = jnp.zeros((NP, CNT_W), jnp.float32)
    zeros_s = jnp.zeros((NP, DH), jnp.float32)

    cnt = _count_call(e2, ones_c, zeros_c)[:, :N_NODES]

    p8, q8, g0 = pl.pallas_call(
        _prep_body,
        grid=(N_BLOCKS,),
        in_specs=[pl.BlockSpec((2, ROW_BLOCK, CNT_W), lambda i: (0, i, 0)),
                  _row_spec()],
        out_specs=[_pq_spec, _pq_spec, _cat_spec],
        out_shape=[_pq_shape, _pq_shape, _cat_shape],
    )(cnt, node_feat)

    s1 = _spmm_call(g0.reshape(2 * NP, DH), e2p, zeros_s)[:, :N_NODES]

    h1, g1 = pl.pallas_call(
        _mid_body,
        grid=(N_BLOCKS,),
        in_specs=[pl.BlockSpec((2, ROW_BLOCK, DH), lambda i: (0, i, 0)),
                  _pq_spec, _pq_spec],
        out_specs=[_row_spec(), _cat_spec],
        out_shape=[jax.ShapeDtypeStruct((N_NODES, D), jnp.float32),
                   _cat_shape],
    )(s1, p8, q8)

    s2 = _spmm_call(g1.reshape(2 * NP, DH), e2p, zeros_s)[:, :N_NODES]

    full_spec = pl.BlockSpec((D, D), lambda i: (0, 0))
    vec_spec = pl.BlockSpec((1, D), lambda i: (0, 0))
    pre, stats = pl.pallas_call(
        _pre_stats_body,
        grid=(N_BLOCKS,),
        in_specs=[_row_spec(), _row_spec(),
                  pl.BlockSpec((2, ROW_BLOCK, DH), lambda i: (0, i, 0)),
                  _pq_spec, full_spec, full_spec, full_spec, vec_spec],
        out_specs=[_row_spec(), pl.BlockSpec((2, D), lambda i: (0, 0))],
        out_shape=[jax.ShapeDtypeStruct((N_NODES, D), jnp.float32),
                   jax.ShapeDtypeStruct((2, D), jnp.float32)],
        scratch_shapes=[pltpu.VMEM((2, D), jnp.float32)],
    )(node_feat, h1, s2, q8, W0, W1, W2, bias.reshape(1, D))

    out = pl.pallas_call(
        _bn_apply_body,
        grid=(N_BLOCKS,),
        in_specs=[_row_spec(), pl.BlockSpec((2, D), lambda i: (0, 0)),
                  _row_spec(), vec_spec, vec_spec],
        out_specs=_row_spec(),
        out_shape=jax.ShapeDtypeStruct((N_NODES, D), jnp.float32),
    )(pre, stats, node_feat, gamma.reshape(1, D), beta.reshape(1, D))
    return out


# speedup vs baseline: 1.1528x; 1.1528x over previous
import functools
import jax
import jax.numpy as jnp
from jax import lax
from jax.experimental import pallas as pl
from jax.experimental.pallas import tpu as pltpu
from jax.experimental.pallas import tpu_sc as plsc

N_NODES = 10000
N_EDGES = 160000
D = 256
DH = 128

ROW_BLOCK = 1000
N_BLOCKS = N_NODES // ROW_BLOCK

EC = 128
E_ROWS = N_EDGES // EC
N_SUB = 16
NP = 10112
ROWS_PER_TILE = NP // N_SUB
CNT_W = 8
NLANE = 16
E_ROWS_P = 1280
CH_PER_TILE = E_ROWS_P // N_SUB
T_ITER = CH_PER_TILE // 2

_sc_mesh = plsc.VectorSubcoreMesh(core_axis_name="c", subcore_axis_name="s",
                                  num_cores=2, num_subcores=N_SUB)


def _count_body(e2, ones_hbm, zeros_hbm, out_hbm, acc_sh, idx_v, ones_v):
    c = lax.axis_index("c")
    sid = lax.axis_index("s")
    stripe = pl.ds(sid * ROWS_PER_TILE, ROWS_PER_TILE)
    pltpu.sync_copy(ones_hbm, ones_v)
    pltpu.sync_copy(zeros_hbm.at[stripe], acc_sh.at[stripe])
    plsc.subcore_barrier()

    n_iter = (E_ROWS - sid + N_SUB - 1) // N_SUB

    def body(j, carry):
        r = sid + j * N_SUB
        pltpu.sync_copy(e2.at[c, r], idx_v)
        pltpu.sync_copy(ones_v, acc_sh.at[idx_v], add=True)
        return carry

    lax.fori_loop(0, n_iter, body, 0)
    plsc.subcore_barrier()
    pltpu.sync_copy(acc_sh.at[stripe], out_hbm.at[c, stripe])


_count_call = functools.partial(
    pl.kernel,
    out_type=jax.ShapeDtypeStruct((2, NP, CNT_W), jnp.float32),
    mesh=_sc_mesh,
    compiler_params=pltpu.CompilerParams(use_tc_tiling_on_sc=False),
    scratch_types=[
        pltpu.VMEM_SHARED((NP, CNT_W), jnp.float32),
        pltpu.VMEM((EC,), jnp.int32),
        pltpu.VMEM((EC, CNT_W), jnp.float32),
    ],
)(_count_body)


def _spmm_body(gcat, e2, zeros_hbm, out_cat,
               acc_sh, src_v, dst_v, idx2_v, buf, sem):
    c = lax.axis_index("c")
    sid = lax.axis_index("s")
    stripe = pl.ds(sid * ROWS_PER_TILE, ROWS_PER_TILE)
    pltpu.sync_copy(zeros_hbm.at[stripe], acc_sh.at[stripe])
    plsc.subcore_barrier()

    coff = c * NP
    n_iter = (E_ROWS - sid + N_SUB - 1) // N_SUB

    def body(j, carry):
        r = sid + j * N_SUB
        pltpu.sync_copy(e2.at[0, r], src_v)
        pltpu.sync_copy(e2.at[1, r], dst_v)
        for k in range(EC // NLANE):
            sl = pl.ds(k * NLANE, NLANE)
            idx2_v[sl] = src_v[sl] + coff
        pltpu.async_copy(gcat.at[idx2_v], buf, sem).wait()
        pltpu.sync_copy(buf, acc_sh.at[dst_v], add=True)
        return carry

    lax.fori_loop(0, n_iter, body, 0)
    plsc.subcore_barrier()
    pltpu.sync_copy(acc_sh.at[stripe], out_cat.at[c, stripe])


_spmm_call = functools.partial(
    pl.kernel,
    out_type=jax.ShapeDtypeStruct((2, NP, DH), jnp.float32),
    mesh=_sc_mesh,
    scratch_types=[
        pltpu.VMEM_SHARED((NP, DH), jnp.float32),
        pltpu.VMEM((EC,), jnp.int32),
        pltpu.VMEM((EC,), jnp.int32),
        pltpu.VMEM((EC,), jnp.int32),
        pltpu.VMEM((EC, DH), jnp.float32),
        pltpu.SemaphoreType.DMA,
    ],
)(_spmm_body)


def _prep_body(cnt_ref, nf_ref, p_ref, q_ref, g_ref):
    cs = cnt_ref[0, :, 0:1]
    cd = cnt_ref[1, :, 0:1]
    p = jnp.where(cs > 0, lax.rsqrt(cs), 0.0)
    q = jnp.where(cd > 0, lax.rsqrt(cd), 0.0)
    p_ref[...] = jnp.broadcast_to(p, p_ref.shape)
    q_ref[...] = jnp.broadcast_to(q, q_ref.shape)
    g = nf_ref[...] * p
    g_ref[0] = g[:, :DH]
    g_ref[1] = g[:, DH:]


def _mid_body(s_ref, p_ref, q_ref, h_ref, g_ref):
    p = p_ref[:, 0:1]
    q = q_ref[:, 0:1]
    hlo = s_ref[0] * q
    hhi = s_ref[1] * q
    h_ref[...] = jnp.concatenate([hlo, hhi], axis=1)
    g_ref[0] = hlo * p
    g_ref[1] = hhi * p


def _final_body(h0_ref, h1_ref, s2_ref, q_ref,
                w0_ref, w1_ref, w2_ref, b_ref, g_ref, be_ref,
                out_ref, pre_scr, acc_ref):
    ph = pl.program_id(0)
    i = pl.program_id(1)
    blk = pl.ds(i * ROW_BLOCK, ROW_BLOCK)

    @pl.when((ph == 0) & (i == 0))
    def _():
        acc_ref[...] = jnp.zeros_like(acc_ref)

    @pl.when(ph == 0)
    def _():
        q = q_ref[:, 0:1]
        h2 = jnp.concatenate([s2_ref[0] * q, s2_ref[1] * q], axis=1)
        dn = (((1,), (1,)), ((), ()))
        pre = lax.dot_general(h0_ref[...], w0_ref[...], dn,
                              preferred_element_type=jnp.float32)
        pre += lax.dot_general(h1_ref[...], w1_ref[...], dn,
                               preferred_element_type=jnp.float32)
        pre += lax.dot_general(h2, w2_ref[...], dn,
                               preferred_element_type=jnp.float32)
        pre += b_ref[...]
        pre_scr[blk, :] = pre
        acc_ref[0, :] += jnp.sum(pre, axis=0)
        acc_ref[1, :] += jnp.sum(pre * pre, axis=0)
        out_ref[...] = pre

    @pl.when(ph == 1)
    def _():
        mean = acc_ref[0, :] / N_NODES
        var = acc_ref[1, :] / N_NODES - mean * mean
        rstd = lax.rsqrt(var + 1e-5)
        h = (pre_scr[blk, :] - mean) * (rstd * g_ref[...]) + be_ref[...]
        out_ref[...] = h0_ref[...] + jnp.maximum(h, 0.0)


def _row_spec(w=D):
    return pl.BlockSpec((ROW_BLOCK, w), lambda i: (i, 0))


_cat_spec = pl.BlockSpec((2, ROW_BLOCK, DH), lambda i: (0, i, 0))
_pq_spec = pl.BlockSpec((ROW_BLOCK, CNT_W), lambda i: (i, 0))
_cat_shape = jax.ShapeDtypeStruct((2, NP, DH), jnp.float32)
_pq_shape = jax.ShapeDtypeStruct((N_NODES, CNT_W), jnp.float32)


@jax.jit
def kernel(node_feat, edge_index, W0, W1, W2, bias, gamma, beta):
    e2 = edge_index.reshape(2, E_ROWS, EC)
    ones_c = jnp.ones((EC, CNT_W), jnp.float32)
    zeros_c = jnp.zeros((NP, CNT_W), jnp.float32)
    zeros_s = jnp.zeros((NP, DH), jnp.float32)

    cnt = _count_call(e2, ones_c, zeros_c)[:, :N_NODES]

    p8, q8, g0 = pl.pallas_call(
        _prep_body,
        grid=(N_BLOCKS,),
        in_specs=[pl.BlockSpec((2, ROW_BLOCK, CNT_W), lambda i: (0, i, 0)),
                  _row_spec()],
        out_specs=[_pq_spec, _pq_spec, _cat_spec],
        out_shape=[_pq_shape, _pq_shape, _cat_shape],
    )(cnt, node_feat)

    s1 = _spmm_call(g0.reshape(2 * NP, DH), e2, zeros_s)[:, :N_NODES]

    h1, g1 = pl.pallas_call(
        _mid_body,
        grid=(N_BLOCKS,),
        in_specs=[pl.BlockSpec((2, ROW_BLOCK, DH), lambda i: (0, i, 0)),
                  _pq_spec, _pq_spec],
        out_specs=[_row_spec(), _cat_spec],
        out_shape=[jax.ShapeDtypeStruct((N_NODES, D), jnp.float32),
                   _cat_shape],
    )(s1, p8, q8)

    s2 = _spmm_call(g1.reshape(2 * NP, DH), e2, zeros_s)[:, :N_NODES]

    row2 = pl.BlockSpec((ROW_BLOCK, D), lambda ph, i: (i, 0))
    full2 = pl.BlockSpec((D, D), lambda ph, i: (0, 0))
    vec2 = pl.BlockSpec((1, D), lambda ph, i: (0, 0))
    out = pl.pallas_call(
        _final_body,
        grid=(2, N_BLOCKS),
        in_specs=[row2, row2,
                  pl.BlockSpec((2, ROW_BLOCK, DH), lambda ph, i: (0, i, 0)),
                  pl.BlockSpec((ROW_BLOCK, CNT_W), lambda ph, i: (i, 0)),
                  full2, full2, full2, vec2, vec2, vec2],
        out_specs=row2,
        out_shape=jax.ShapeDtypeStruct((N_NODES, D), jnp.float32),
        scratch_shapes=[pltpu.VMEM((N_NODES, D), jnp.float32),
                        pltpu.VMEM((2, D), jnp.float32)],
        compiler_params=pltpu.CompilerParams(
            vmem_limit_bytes=48 * 1024 * 1024),
    )(node_feat, h1, s2, q8, W0, W1, W2, bias.reshape(1, D),
      gamma.reshape(1, D), beta.reshape(1, D))
    return out


# speedup vs baseline: 1.6236x; 1.4083x over previous
import functools
import jax
import jax.numpy as jnp
from jax import lax
from jax.experimental import pallas as pl
from jax.experimental.pallas import tpu as pltpu
from jax.experimental.pallas import tpu_sc as plsc

N_NODES = 10000
N_EDGES = 160000
D = 256
DH = 128

ROW_BLOCK = 1000
N_BLOCKS = N_NODES // ROW_BLOCK

EC = 128
E_ROWS = N_EDGES // EC
N_SUB = 16
NP = 10112
ROWS_PER_TILE = NP // N_SUB
CNT_W = 8
NLANE = 16
E_ROWS_P = 1280
CH_PER_TILE = E_ROWS_P // N_SUB
T_ITER = CH_PER_TILE // 2

_sc_mesh = plsc.VectorSubcoreMesh(core_axis_name="c", subcore_axis_name="s",
                                  num_cores=2, num_subcores=N_SUB)


def _count_body(e2, ones_hbm, zeros_hbm, out_hbm, acc_sh, idx_v, ones_v):
    c = lax.axis_index("c")
    sid = lax.axis_index("s")
    stripe = pl.ds(sid * ROWS_PER_TILE, ROWS_PER_TILE)
    pltpu.sync_copy(ones_hbm, ones_v)
    pltpu.sync_copy(zeros_hbm.at[stripe], acc_sh.at[stripe])
    plsc.subcore_barrier()

    n_iter = (E_ROWS - sid + N_SUB - 1) // N_SUB

    def body(j, carry):
        r = sid + j * N_SUB
        pltpu.sync_copy(e2.at[c, r], idx_v)
        pltpu.sync_copy(ones_v, acc_sh.at[idx_v], add=True)
        return carry

    lax.fori_loop(0, n_iter, body, 0)
    plsc.subcore_barrier()
    pltpu.sync_copy(acc_sh.at[stripe], out_hbm.at[c, stripe])


_count_call = functools.partial(
    pl.kernel,
    out_type=jax.ShapeDtypeStruct((2, NP, CNT_W), jnp.float32),
    mesh=_sc_mesh,
    compiler_params=pltpu.CompilerParams(use_tc_tiling_on_sc=False),
    scratch_types=[
        pltpu.VMEM_SHARED((NP, CNT_W), jnp.float32),
        pltpu.VMEM((EC,), jnp.int32),
        pltpu.VMEM((EC, CNT_W), jnp.float32),
    ],
)(_count_body)


def _spmm_body(gcat, e2, zeros_hbm, out_cat,
               acc_sh, idx2_v, dst_v, idx1_v, dst1_v, buf, buf1, sg0, sg1):
    c = lax.axis_index("c")
    sid = lax.axis_index("s")
    stripe = pl.ds(sid * ROWS_PER_TILE, ROWS_PER_TILE)
    pltpu.sync_copy(zeros_hbm.at[stripe], acc_sh.at[stripe])
    plsc.subcore_barrier()

    coff = c * NP

    def addoff(ref):
        for k in range(EC // NLANE):
            sl = pl.ds(k * NLANE, NLANE)
            ref[sl] = ref[sl] + coff

    def load_idx(r, idx_ref, dst_ref):
        pltpu.sync_copy(e2.at[0, r], idx_ref)
        pltpu.sync_copy(e2.at[1, r], dst_ref)
        addoff(idx_ref)

    n_iter = (E_ROWS - sid + N_SUB - 1) // N_SUB
    n_pairs = n_iter // 2

    load_idx(sid, idx2_v, dst_v)
    pltpu.async_copy(gcat.at[idx2_v], buf, sg0)

    def body(t, carry):
        a = sid + 2 * t * N_SUB
        b = a + N_SUB
        load_idx(b, idx1_v, dst1_v)
        pltpu.async_copy(gcat.at[idx1_v], buf1, sg1)
        pltpu.make_async_copy(gcat.at[idx2_v], buf, sg0).wait()
        pltpu.sync_copy(buf, acc_sh.at[dst_v], add=True)

        @pl.when(t < n_pairs - 1)
        def _():
            load_idx(b + N_SUB, idx2_v, dst_v)
            pltpu.async_copy(gcat.at[idx2_v], buf, sg0)

        pltpu.make_async_copy(gcat.at[idx1_v], buf1, sg1).wait()
        pltpu.sync_copy(buf1, acc_sh.at[dst1_v], add=True)
        return carry

    lax.fori_loop(0, n_pairs, body, 0)

    @pl.when(n_iter % 2 == 1)
    def _():
        r_last = sid + (n_iter - 1) * N_SUB
        load_idx(r_last, idx2_v, dst_v)
        pltpu.async_copy(gcat.at[idx2_v], buf, sg0).wait()
        pltpu.sync_copy(buf, acc_sh.at[dst_v], add=True)

    plsc.subcore_barrier()
    pltpu.sync_copy(acc_sh.at[stripe], out_cat.at[c, stripe])


_spmm_call = functools.partial(
    pl.kernel,
    out_type=jax.ShapeDtypeStruct((2, NP, DH), jnp.float32),
    mesh=_sc_mesh,
    scratch_types=[
        pltpu.VMEM_SHARED((NP, DH), jnp.float32),
        pltpu.VMEM((EC,), jnp.int32),
        pltpu.VMEM((EC,), jnp.int32),
        pltpu.VMEM((EC,), jnp.int32),
        pltpu.VMEM((EC,), jnp.int32),
        pltpu.VMEM((EC, DH), jnp.float32),
        pltpu.VMEM((EC, DH), jnp.float32),
        pltpu.SemaphoreType.DMA,
        pltpu.SemaphoreType.DMA,
    ],
)(_spmm_body)


def _prep_body(cnt_ref, nf_ref, p_ref, q_ref, g_ref):
    cs = cnt_ref[0, :, 0:1]
    cd = cnt_ref[1, :, 0:1]
    p = jnp.where(cs > 0, lax.rsqrt(cs), 0.0)
    q = jnp.where(cd > 0, lax.rsqrt(cd), 0.0)
    p_ref[...] = jnp.broadcast_to(p, p_ref.shape)
    q_ref[...] = jnp.broadcast_to(q, q_ref.shape)
    g = nf_ref[...] * p
    g_ref[0] = g[:, :DH]
    g_ref[1] = g[:, DH:]


def _mid_body(s_ref, p_ref, q_ref, h_ref, g_ref):
    p = p_ref[:, 0:1]
    q = q_ref[:, 0:1]
    hlo = s_ref[0] * q
    hhi = s_ref[1] * q
    h_ref[...] = jnp.concatenate([hlo, hhi], axis=1)
    g_ref[0] = hlo * p
    g_ref[1] = hhi * p


def _final_body(h0_ref, h1_ref, s2_ref, q_ref,
                w0_ref, w1_ref, w2_ref, b_ref, g_ref, be_ref,
                out_ref, pre_scr, acc_ref):
    ph = pl.program_id(0)
    i = pl.program_id(1)
    blk = pl.ds(i * ROW_BLOCK, ROW_BLOCK)

    @pl.when((ph == 0) & (i == 0))
    def _():
        acc_ref[...] = jnp.zeros_like(acc_ref)

    @pl.when(ph == 0)
    def _():
        q = q_ref[:, 0:1]
        h2 = jnp.concatenate([s2_ref[0] * q, s2_ref[1] * q], axis=1)
        dn = (((1,), (1,)), ((), ()))
        pre = lax.dot_general(h0_ref[...], w0_ref[...], dn,
                              preferred_element_type=jnp.float32)
        pre += lax.dot_general(h1_ref[...], w1_ref[...], dn,
                               preferred_element_type=jnp.float32)
        pre += lax.dot_general(h2, w2_ref[...], dn,
                               preferred_element_type=jnp.float32)
        pre += b_ref[...]
        pre_scr[blk, :] = pre
        acc_ref[0, :] += jnp.sum(pre, axis=0)
        acc_ref[1, :] += jnp.sum(pre * pre, axis=0)
        out_ref[...] = pre

    @pl.when(ph == 1)
    def _():
        mean = acc_ref[0, :] / N_NODES
        var = acc_ref[1, :] / N_NODES - mean * mean
        rstd = lax.rsqrt(var + 1e-5)
        h = (pre_scr[blk, :] - mean) * (rstd * g_ref[...]) + be_ref[...]
        out_ref[...] = h0_ref[...] + jnp.maximum(h, 0.0)


def _row_spec(w=D):
    return pl.BlockSpec((ROW_BLOCK, w), lambda i: (i, 0))


_cat_spec = pl.BlockSpec((2, ROW_BLOCK, DH), lambda i: (0, i, 0))
_pq_spec = pl.BlockSpec((ROW_BLOCK, CNT_W), lambda i: (i, 0))
_cat_shape = jax.ShapeDtypeStruct((2, NP, DH), jnp.float32)
_pq_shape = jax.ShapeDtypeStruct((N_NODES, CNT_W), jnp.float32)


@jax.jit
def kernel(node_feat, edge_index, W0, W1, W2, bias, gamma, beta):
    e2 = edge_index.reshape(2, E_ROWS, EC)
    ones_c = jnp.ones((EC, CNT_W), jnp.float32)
    zeros_c = jnp.zeros((NP, CNT_W), jnp.float32)
    zeros_s = jnp.zeros((NP, DH), jnp.float32)

    cnt = _count_call(e2, ones_c, zeros_c)[:, :N_NODES]

    p8, q8, g0 = pl.pallas_call(
        _prep_body,
        grid=(N_BLOCKS,),
        in_specs=[pl.BlockSpec((2, ROW_BLOCK, CNT_W), lambda i: (0, i, 0)),
                  _row_spec()],
        out_specs=[_pq_spec, _pq_spec, _cat_spec],
        out_shape=[_pq_shape, _pq_shape, _cat_shape],
    )(cnt, node_feat)

    s1 = _spmm_call(g0.reshape(2 * NP, DH), e2, zeros_s)[:, :N_NODES]

    h1, g1 = pl.pallas_call(
        _mid_body,
        grid=(N_BLOCKS,),
        in_specs=[pl.BlockSpec((2, ROW_BLOCK, DH), lambda i: (0, i, 0)),
                  _pq_spec, _pq_spec],
        out_specs=[_row_spec(), _cat_spec],
        out_shape=[jax.ShapeDtypeStruct((N_NODES, D), jnp.float32),
                   _cat_shape],
    )(s1, p8, q8)

    s2 = _spmm_call(g1.reshape(2 * NP, DH), e2, zeros_s)[:, :N_NODES]

    row2 = pl.BlockSpec((ROW_BLOCK, D), lambda ph, i: (i, 0))
    full2 = pl.BlockSpec((D, D), lambda ph, i: (0, 0))
    vec2 = pl.BlockSpec((1, D), lambda ph, i: (0, 0))
    out = pl.pallas_call(
        _final_body,
        grid=(2, N_BLOCKS),
        in_specs=[row2, row2,
                  pl.BlockSpec((2, ROW_BLOCK, DH), lambda ph, i: (0, i, 0)),
                  pl.BlockSpec((ROW_BLOCK, CNT_W), lambda ph, i: (i, 0)),
                  full2, full2, full2, vec2, vec2, vec2],
        out_specs=row2,
        out_shape=jax.ShapeDtypeStruct((N_NODES, D), jnp.float32),
        scratch_shapes=[pltpu.VMEM((N_NODES, D), jnp.float32),
                        pltpu.VMEM((2, D), jnp.float32)],
        compiler_params=pltpu.CompilerParams(
            vmem_limit_bytes=48 * 1024 * 1024),
    )(node_feat, h1, s2, q8, W0, W1, W2, bias.reshape(1, D),
      gamma.reshape(1, D), beta.reshape(1, D))
    return out
